# baseline (speedup 1.0000x reference)
import math
from functools import partial

import numpy as np
import jax
import jax.numpy as jnp
from jax.experimental import pallas as pl
from jax.experimental.pallas import tpu as pltpu

_INV_SQRT2 = 1.0 / math.sqrt(2.0)


def _dft_consts(H, W, m1, m2):
    k1 = np.concatenate([np.arange(m1), np.arange(H - m1, H)]).astype(np.float64)
    k2 = np.arange(m2, dtype=np.float64)
    h = np.arange(H, dtype=np.float64)
    w = np.arange(W, dtype=np.float64)

    fscale = 1.0 / math.sqrt(H * W)

    ang_fh = 2.0 * np.pi * np.outer(k1, h) / H
    fhr = np.cos(ang_fh) * fscale
    fhi = -np.sin(ang_fh) * fscale
    ang_fw = 2.0 * np.pi * np.outer(w, k2) / W
    gfr = np.cos(ang_fw)
    gfi = -np.sin(ang_fw)

    ang_eh = 2.0 * np.pi * np.outer(h, k1) / H
    ehr = np.cos(ang_eh)
    ehi = np.sin(ang_eh)
    c = np.where(np.logical_or(k2 == 0,
                               np.logical_and(W % 2 == 0, k2 == (W // 2))),
                 1.0, 2.0)
    ang_gw = 2.0 * np.pi * np.outer(k2, w) / W
    iscale = 1.0 / (H * W)
    gwr = np.cos(ang_gw) * c[:, None] * iscale
    gwi = np.sin(ang_gw) * c[:, None] * iscale

    G = np.concatenate([gfr, gfi], axis=1)
    FHT = np.concatenate([fhr, fhi], axis=0).T
    EH_r = np.concatenate([ehr.T, -ehi.T], axis=0)
    EH_i = np.concatenate([ehi.T, ehr.T], axis=0)
    Gw_cat = np.concatenate([gwr, -gwi], axis=0)

    f32 = lambda a: jnp.asarray(a, dtype=jnp.float32)
    return f32(G), f32(FHT), f32(EH_r), f32(EH_i), f32(Gw_cat)


def _wprep_kernel(w1r_ref, w1i_ref, w2r_ref, w2i_ref,
                  wr_ref, wsum_ref, wdif_ref):
    _, co, m1, m2 = w1r_ref.shape
    K = m2 * 2 * m1
    wr = jnp.concatenate([w1r_ref[0].transpose(0, 2, 1),
                          w2r_ref[0].transpose(0, 2, 1)],
                         axis=2).reshape(co, K)
    wi = jnp.concatenate([w1i_ref[0].transpose(0, 2, 1),
                          w2i_ref[0].transpose(0, 2, 1)],
                         axis=2).reshape(co, K)
    wr_ref[0] = wr
    wsum_ref[0] = wr + wi
    wdif_ref[0] = wi - wr


def _fused_kernel(x_ref, g_ref, fht_ref, ehr_ref, ehi_ref, gw_ref,
                  wr_s, wsum_s, wdif_s, wlin_ref, b_ref,
                  o_ref, *, m1, m2):
    ci, H, W = x_ref.shape[1], x_ref.shape[2], x_ref.shape[3]
    co = o_ref.shape[1]
    two_m1 = 2 * m1
    K = m2 * two_m1

    xb = x_ref[0]

    A = jnp.dot(xb.reshape(ci * H, W), g_ref[...],
                preferred_element_type=jnp.float32)
    A = A.reshape(ci, H, 2 * m2).transpose(0, 2, 1)
    P = jnp.dot(A.reshape(ci * 2 * m2, H), fht_ref[...],
                preferred_element_type=jnp.float32)
    P = P.reshape(ci, 2 * m2, 2 * two_m1)

    xr = (P[:, :m2, :two_m1] - P[:, m2:, two_m1:]).reshape(ci, K)
    xi = (P[:, m2:, :two_m1] + P[:, :m2, two_m1:]).reshape(ci, K)

    kt = wr_s[...] * (xr + xi)[:, None, :]
    yr = jnp.sum(kt - xi[:, None, :] * wsum_s[...], axis=0)
    yi = jnp.sum(kt + xr[:, None, :] * wdif_s[...], axis=0)

    ycat = jnp.concatenate([yr.reshape(co, m2, two_m1),
                            yi.reshape(co, m2, two_m1)], axis=2)
    ycat2 = ycat.reshape(co * m2, 2 * two_m1)
    pr = jnp.dot(ycat2, ehr_ref[...],
                 preferred_element_type=jnp.float32)
    pi = jnp.dot(ycat2, ehi_ref[...],
                 preferred_element_type=jnp.float32)
    pboth = jnp.concatenate([pr.reshape(co, m2, H),
                             pi.reshape(co, m2, H)], axis=1)
    pboth = pboth.transpose(0, 2, 1)
    y_spec = jnp.dot(pboth.reshape(co * H, 2 * m2), gw_ref[...],
                     preferred_element_type=jnp.float32)
    y_spec = y_spec.reshape(co, H, W)

    res = jnp.dot(wlin_ref[...], xb.reshape(ci, H * W),
                  preferred_element_type=jnp.float32)
    z = y_spec + res.reshape(co, H, W) + b_ref[...][:, :, None]
    o_ref[0] = 0.5 * z * (1.0 + jax.lax.erf(z * _INV_SQRT2))


def kernel(x, w_lin, b_lin, w1r, w1i, w2r, w2i):
    B, Ci, H, W = x.shape
    Co = w_lin.shape[0]
    m1, m2 = w1r.shape[2], w1r.shape[3]
    two_m1 = 2 * m1
    K = m2 * two_m1

    G, FHT, EH_r, EH_i, Gw_cat = _dft_consts(H, W, m1, m2)

    f32 = jnp.float32

    wshape = jax.ShapeDtypeStruct((Ci, Co, K), f32)
    wspec = pl.BlockSpec((1, Co, m1, m2), lambda c: (c, 0, 0, 0))
    ospec = pl.BlockSpec((1, Co, K), lambda c: (c, 0, 0))
    wr_m, wsum, wdif = pl.pallas_call(
        _wprep_kernel,
        out_shape=(wshape, wshape, wshape),
        grid=(Ci,),
        in_specs=[wspec, wspec, wspec, wspec],
        out_specs=(ospec, ospec, ospec),
        compiler_params=pltpu.CompilerParams(
            dimension_semantics=("arbitrary",)),
    )(w1r.astype(f32), w1i.astype(f32), w2r.astype(f32), w2i.astype(f32))

    const = lambda b: (0, 0)
    const3 = lambda b: (0, 0, 0)

    out = pl.pallas_call(
        partial(_fused_kernel, m1=m1, m2=m2),
        out_shape=jax.ShapeDtypeStruct((B, Co, H, W), f32),
        grid=(B,),
        in_specs=[
            pl.BlockSpec((1, Ci, H, W), lambda b: (b, 0, 0, 0)),
            pl.BlockSpec((W, 2 * m2), const),
            pl.BlockSpec((H, 2 * two_m1), const),
            pl.BlockSpec((2 * two_m1, H), const),
            pl.BlockSpec((2 * two_m1, H), const),
            pl.BlockSpec((2 * m2, W), const),
            pl.BlockSpec((Ci, Co, K), const3),
            pl.BlockSpec((Ci, Co, K), const3),
            pl.BlockSpec((Ci, Co, K), const3),
            pl.BlockSpec((Co, Ci), const),
            pl.BlockSpec((Co, 1), const),
        ],
        out_specs=pl.BlockSpec((1, Co, H, W), lambda b: (b, 0, 0, 0)),
        compiler_params=pltpu.CompilerParams(
            dimension_semantics=("arbitrary",)),
    )(x.astype(f32), G, FHT, EH_r, EH_i, Gw_cat,
      wr_m, wsum, wdif, w_lin.astype(f32), b_lin.astype(f32))
    return out


# speedup vs baseline: 1.2898x; 1.2898x over previous
import math
from functools import partial

import numpy as np
import jax
import jax.numpy as jnp
from jax.experimental import pallas as pl
from jax.experimental.pallas import tpu as pltpu

_INV_SQRT2 = 1.0 / math.sqrt(2.0)


def _dft_consts(H, W, m1, m2):
    k1 = np.concatenate([np.arange(m1), np.arange(H - m1, H)]).astype(np.float64)
    k2 = np.arange(m2, dtype=np.float64)
    h = np.arange(H, dtype=np.float64)
    w = np.arange(W, dtype=np.float64)

    fscale = 1.0 / math.sqrt(H * W)

    ang_fh = 2.0 * np.pi * np.outer(k1, h) / H
    fhr = np.cos(ang_fh) * fscale
    fhi = -np.sin(ang_fh) * fscale
    ang_fw = 2.0 * np.pi * np.outer(w, k2) / W
    gfr = np.cos(ang_fw)
    gfi = -np.sin(ang_fw)

    ang_eh = 2.0 * np.pi * np.outer(h, k1) / H
    ehr = np.cos(ang_eh)
    ehi = np.sin(ang_eh)
    c = np.where(np.logical_or(k2 == 0,
                               np.logical_and(W % 2 == 0, k2 == (W // 2))),
                 1.0, 2.0)
    ang_gw = 2.0 * np.pi * np.outer(k2, w) / W
    iscale = 1.0 / (H * W)
    gwr = np.cos(ang_gw) * c[:, None] * iscale
    gwi = np.sin(ang_gw) * c[:, None] * iscale

    G = np.concatenate([gfr, gfi], axis=1)
    FH = np.concatenate([fhr, fhi], axis=0)
    EH_r = np.concatenate([ehr.T, -ehi.T], axis=0)
    EH_i = np.concatenate([ehi.T, ehr.T], axis=0)
    Gw_cat = np.concatenate([gwr, -gwi], axis=0)

    f32 = lambda a: jnp.asarray(a, dtype=jnp.float32)
    return f32(G), f32(FH), f32(EH_r), f32(EH_i), f32(Gw_cat)


def _fused_kernel(x_ref, g_ref, fh_ref, ehr_ref, ehi_ref, gw_ref,
                  w1r_ref, w1i_ref, w2r_ref, w2i_ref, wlin_ref, b_ref,
                  o_ref, *, m1, m2):
    ci, H, W = x_ref.shape[1], x_ref.shape[2], x_ref.shape[3]
    co = o_ref.shape[1]
    two_m1 = 2 * m1
    Kh = m1 * m2

    xb = x_ref[0]

    A = jnp.dot(xb.reshape(ci * H, W), g_ref[...],
                preferred_element_type=jnp.float32)
    A = A.reshape(ci, H, 2 * m2)
    P = jax.lax.dot_general(fh_ref[...], A, (((1,), (1,)), ((), ())),
                            preferred_element_type=jnp.float32)
    P = P.transpose(1, 0, 2)

    xr = (P[:, :two_m1, :m2] - P[:, two_m1:, m2:]).reshape(ci, 2 * Kh)
    xi = (P[:, two_m1:, :m2] + P[:, :two_m1, m2:]).reshape(ci, 2 * Kh)

    xr1 = xr[:, None, :Kh]
    xr2 = xr[:, None, Kh:]
    xi1 = xi[:, None, :Kh]
    xi2 = xi[:, None, Kh:]
    w1r, w1i = w1r_ref[...], w1i_ref[...]
    w2r, w2i = w2r_ref[...], w2i_ref[...]
    yr1 = jnp.sum(xr1 * w1r - xi1 * w1i, axis=0)
    yi1 = jnp.sum(xr1 * w1i + xi1 * w1r, axis=0)
    yr2 = jnp.sum(xr2 * w2r - xi2 * w2i, axis=0)
    yi2 = jnp.sum(xr2 * w2i + xi2 * w2r, axis=0)

    ycat = jnp.concatenate([yr1.reshape(co, m1, m2),
                            yr2.reshape(co, m1, m2),
                            yi1.reshape(co, m1, m2),
                            yi2.reshape(co, m1, m2)], axis=1)
    pr = jax.lax.dot_general(ycat, ehr_ref[...], (((1,), (0,)), ((), ())),
                             preferred_element_type=jnp.float32)
    pi = jax.lax.dot_general(ycat, ehi_ref[...], (((1,), (0,)), ((), ())),
                             preferred_element_type=jnp.float32)
    pboth = jnp.concatenate([pr, pi], axis=1)
    pboth = pboth.transpose(0, 2, 1)
    y_spec = jnp.dot(pboth.reshape(co * H, 2 * m2), gw_ref[...],
                     preferred_element_type=jnp.float32)
    y_spec = y_spec.reshape(co, H, W)

    res = jnp.dot(wlin_ref[...], xb.reshape(ci, H * W),
                  preferred_element_type=jnp.float32)
    z = y_spec + res.reshape(co, H, W) + b_ref[...][:, :, None]
    o_ref[0] = 0.5 * z * (1.0 + jax.lax.erf(z * _INV_SQRT2))


def kernel(x, w_lin, b_lin, w1r, w1i, w2r, w2i):
    B, Ci, H, W = x.shape
    Co = w_lin.shape[0]
    m1, m2 = w1r.shape[2], w1r.shape[3]
    two_m1 = 2 * m1
    Kh = m1 * m2

    G, FH, EH_r, EH_i, Gw_cat = _dft_consts(H, W, m1, m2)

    f32 = jnp.float32
    wargs = [a.astype(f32).reshape(Ci, Co, Kh) for a in (w1r, w1i, w2r, w2i)]

    const = lambda b: (0, 0)
    const3 = lambda b: (0, 0, 0)

    out = pl.pallas_call(
        partial(_fused_kernel, m1=m1, m2=m2),
        out_shape=jax.ShapeDtypeStruct((B, Co, H, W), f32),
        grid=(B,),
        in_specs=[
            pl.BlockSpec((1, Ci, H, W), lambda b: (b, 0, 0, 0)),
            pl.BlockSpec((W, 2 * m2), const),
            pl.BlockSpec((2 * two_m1, H), const),
            pl.BlockSpec((2 * two_m1, H), const),
            pl.BlockSpec((2 * two_m1, H), const),
            pl.BlockSpec((2 * m2, W), const),
            pl.BlockSpec((Ci, Co, Kh), const3),
            pl.BlockSpec((Ci, Co, Kh), const3),
            pl.BlockSpec((Ci, Co, Kh), const3),
            pl.BlockSpec((Ci, Co, Kh), const3),
            pl.BlockSpec((Co, Ci), const),
            pl.BlockSpec((Co, 1), const),
        ],
        out_specs=pl.BlockSpec((1, Co, H, W), lambda b: (b, 0, 0, 0)),
        compiler_params=pltpu.CompilerParams(
            dimension_semantics=("arbitrary",)),
    )(x.astype(f32), G, FH, EH_r, EH_i, Gw_cat,
      *wargs, w_lin.astype(f32), b_lin.astype(f32))
    return out
